# baseline (speedup 1.0000x reference)
import functools

import jax
import jax.numpy as jnp
from jax import lax
from jax.experimental import pallas as pl
from jax.experimental.pallas import tpu as pltpu
from jax.experimental.pallas import tpu_sc as plsc

N = 100000
LANES = 16
NUM_CORES = 2
NUM_SUBCORES = 16
NUM_WORKERS = NUM_CORES * NUM_SUBCORES
CHUNK = 3136
LAST_BASE = N - CHUNK
TABLE = 100
TABLE_PAD = 128


def _sc_gather_refs(atoms, table):
    mesh = plsc.VectorSubcoreMesh(core_axis_name="c", subcore_axis_name="s")

    @functools.partial(
        pl.kernel,
        mesh=mesh,
        out_type=jax.ShapeDtypeStruct((N,), jnp.float32),
        compiler_params=pltpu.CompilerParams(needs_layout_passes=False),
        scratch_types=[
            pltpu.VMEM((TABLE_PAD,), jnp.float32),
            pltpu.VMEM((CHUNK,), jnp.int32),
            pltpu.VMEM((CHUNK,), jnp.float32),
            pltpu.SemaphoreType.DMA,
            pltpu.SemaphoreType.DMA,
        ],
    )
    def k(a_hbm, t_hbm, out_hbm, table_v, idx_v, o_v, sem_t, sem_i):
        wid = lax.axis_index("s") * NUM_CORES + lax.axis_index("c")
        base = pl.multiple_of(lax.min(wid * CHUNK, LAST_BASE), 8)
        cp_t = pltpu.async_copy(t_hbm, table_v.at[pl.ds(0, TABLE)], sem_t)
        cp_i = pltpu.async_copy(a_hbm.at[pl.ds(base, CHUNK)], idx_v, sem_i)
        cp_t.wait()
        cp_i.wait()

        @plsc.parallel_loop(0, CHUNK, LANES, unroll=2)
        def _body(off):
            iv = idx_v[pl.ds(off, LANES)]
            o_v[pl.ds(off, LANES)] = plsc.load_gather(table_v, [iv])

        pltpu.sync_copy(o_v, out_hbm.at[pl.ds(base, CHUNK)])

    return k(atoms, table)


def kernel(atomwise_energies, atoms, atom_refs):
    table = atom_refs.reshape(TABLE)
    refs = _sc_gather_refs(atoms.astype(jnp.int32), table)
    return atomwise_energies + refs.reshape(N, 1)


# speedup vs baseline: 1.0368x; 1.0368x over previous
import functools

import jax
import jax.numpy as jnp
from jax import lax
from jax.experimental import pallas as pl
from jax.experimental.pallas import tpu as pltpu
from jax.experimental.pallas import tpu_sc as plsc

N = 100000
LANES = 16
NUM_CORES = 1
NUM_SUBCORES = 16
NUM_WORKERS = NUM_CORES * NUM_SUBCORES
CHUNK = 6256
LAST_BASE = N - CHUNK
TABLE = 100
TABLE_PAD = 128


def _sc_gather_refs(atoms, table):
    mesh = plsc.VectorSubcoreMesh(
        core_axis_name="c", subcore_axis_name="s", num_cores=1)

    @functools.partial(
        pl.kernel,
        mesh=mesh,
        out_type=jax.ShapeDtypeStruct((N,), jnp.float32),
        compiler_params=pltpu.CompilerParams(needs_layout_passes=False),
        scratch_types=[
            pltpu.VMEM((TABLE_PAD,), jnp.float32),
            pltpu.VMEM((CHUNK,), jnp.int32),
            pltpu.VMEM((CHUNK,), jnp.float32),
            pltpu.SemaphoreType.DMA,
            pltpu.SemaphoreType.DMA,
        ],
    )
    def k(a_hbm, t_hbm, out_hbm, table_v, idx_v, o_v, sem_t, sem_i):
        wid = lax.axis_index("s") * NUM_CORES + lax.axis_index("c")
        base = pl.multiple_of(lax.min(wid * CHUNK, LAST_BASE), 8)
        cp_t = pltpu.async_copy(t_hbm, table_v.at[pl.ds(0, TABLE)], sem_t)
        cp_i = pltpu.async_copy(a_hbm.at[pl.ds(base, CHUNK)], idx_v, sem_i)
        cp_t.wait()
        cp_i.wait()

        @plsc.parallel_loop(0, CHUNK, LANES, unroll=2)
        def _body(off):
            iv = idx_v[pl.ds(off, LANES)]
            o_v[pl.ds(off, LANES)] = plsc.load_gather(table_v, [iv])

        pltpu.sync_copy(o_v, out_hbm.at[pl.ds(base, CHUNK)])

    return k(atoms, table)


def kernel(atomwise_energies, atoms, atom_refs):
    table = atom_refs.reshape(TABLE)
    refs = _sc_gather_refs(atoms.astype(jnp.int32), table)
    return atomwise_energies + refs.reshape(N, 1)


# speedup vs baseline: 1.0534x; 1.0160x over previous
import functools

import jax
import jax.numpy as jnp
from jax import lax
from jax.experimental import pallas as pl
from jax.experimental.pallas import tpu as pltpu
from jax.experimental.pallas import tpu_sc as plsc

N = 100000
LANES = 16
NUM_CORES = 1
NUM_SUBCORES = 16
NUM_WORKERS = NUM_CORES * NUM_SUBCORES
CHUNK = 6256
LAST_BASE = N - CHUNK
TABLE = 100
TABLE_PAD = 128


def _sc_gather_refs(atoms, table):
    mesh = plsc.VectorSubcoreMesh(
        core_axis_name="c", subcore_axis_name="s", num_cores=1)

    @functools.partial(
        pl.kernel,
        mesh=mesh,
        out_type=jax.ShapeDtypeStruct((N,), jnp.float32),
        compiler_params=pltpu.CompilerParams(needs_layout_passes=False),
        scratch_types=[
            pltpu.VMEM((TABLE_PAD,), jnp.float32),
            pltpu.VMEM((CHUNK,), jnp.int32),
            pltpu.VMEM((CHUNK,), jnp.float32),
            pltpu.SemaphoreType.DMA,
            pltpu.SemaphoreType.DMA,
        ],
    )
    def k(a_hbm, t_hbm, out_hbm, table_v, idx_v, o_v, sem_t, sem_i):
        wid = lax.axis_index("s") * NUM_CORES + lax.axis_index("c")
        base = pl.multiple_of(lax.min(wid * CHUNK, LAST_BASE), 8)
        cp_t = pltpu.async_copy(t_hbm, table_v.at[pl.ds(0, TABLE)], sem_t)
        cp_i = pltpu.async_copy(a_hbm.at[pl.ds(base, CHUNK)], idx_v, sem_i)
        cp_t.wait()
        cp_i.wait()

        @plsc.parallel_loop(0, CHUNK, LANES, unroll=8)
        def _body(off):
            iv = idx_v[pl.ds(off, LANES)]
            o_v[pl.ds(off, LANES)] = plsc.load_gather(table_v, [iv])

        pltpu.sync_copy(o_v, out_hbm.at[pl.ds(base, CHUNK)])

    return k(atoms, table)


def kernel(atomwise_energies, atoms, atom_refs):
    table = atom_refs.reshape(TABLE)
    refs = _sc_gather_refs(atoms.astype(jnp.int32), table)
    return atomwise_energies + refs.reshape(N, 1)


# speedup vs baseline: 1.0550x; 1.0016x over previous
import functools

import jax
import jax.numpy as jnp
from jax import lax
from jax.experimental import pallas as pl
from jax.experimental.pallas import tpu as pltpu
from jax.experimental.pallas import tpu_sc as plsc

N = 100000
LANES = 16
NUM_CORES = 1
NUM_SUBCORES = 16
NUM_WORKERS = NUM_CORES * NUM_SUBCORES
CHUNK = 6256
LAST_BASE = N - CHUNK
TABLE = 100
TABLE_PAD = 128


def _sc_gather_refs(atoms, table):
    mesh = plsc.VectorSubcoreMesh(
        core_axis_name="c", subcore_axis_name="s", num_cores=1)

    @functools.partial(
        pl.kernel,
        mesh=mesh,
        out_type=jax.ShapeDtypeStruct((N,), jnp.float32),
        compiler_params=pltpu.CompilerParams(needs_layout_passes=False),
        scratch_types=[
            pltpu.VMEM((TABLE_PAD,), jnp.float32),
            pltpu.VMEM((CHUNK,), jnp.int32),
            pltpu.VMEM((CHUNK,), jnp.float32),
            pltpu.SemaphoreType.DMA,
            pltpu.SemaphoreType.DMA,
            pltpu.SemaphoreType.DMA,
            pltpu.SemaphoreType.DMA,
        ],
    )
    def k(a_hbm, t_hbm, out_hbm, table_v, idx_v, o_v, sem_t, sem_i, sem_i2,
          sem_o):
        wid = lax.axis_index("s") * NUM_CORES + lax.axis_index("c")
        base = pl.multiple_of(lax.min(wid * CHUNK, LAST_BASE), 8)
        half = CHUNK // 2
        cp_t = pltpu.async_copy(t_hbm, table_v.at[pl.ds(0, TABLE)], sem_t)
        cp_i = pltpu.async_copy(
            a_hbm.at[pl.ds(base, half)], idx_v.at[pl.ds(0, half)], sem_i)
        cp_i2 = pltpu.async_copy(
            a_hbm.at[pl.ds(base + half, half)], idx_v.at[pl.ds(half, half)],
            sem_i2)
        cp_t.wait()
        cp_i.wait()

        @plsc.parallel_loop(0, half, LANES, unroll=8)
        def _first(off):
            iv = idx_v[pl.ds(off, LANES)]
            o_v[pl.ds(off, LANES)] = plsc.load_gather(table_v, [iv])

        cp_o = pltpu.async_copy(
            o_v.at[pl.ds(0, half)], out_hbm.at[pl.ds(base, half)], sem_o)
        cp_i2.wait()

        @plsc.parallel_loop(half, CHUNK, LANES, unroll=8)
        def _second(off):
            iv = idx_v[pl.ds(off, LANES)]
            o_v[pl.ds(off, LANES)] = plsc.load_gather(table_v, [iv])

        cp_o.wait()
        pltpu.sync_copy(o_v.at[pl.ds(half, half)],
                        out_hbm.at[pl.ds(base + half, half)])

    return k(atoms, table)


def kernel(atomwise_energies, atoms, atom_refs):
    table = atom_refs.reshape(TABLE)
    refs = _sc_gather_refs(atoms.astype(jnp.int32), table)
    return atomwise_energies + refs.reshape(N, 1)
